# Initial kernel scaffold; baseline (speedup 1.0000x reference)
import jax
import jax.numpy as jnp
from jax.experimental import pallas as pl


def kernel(signals, neighbor_actions, W1, att_src1, att_dst1, b1, W2, att_src2, att_dst2, b2, Wq, bq, Wk, bk, Wv, bv, Wo, bo, Wlm, blm, Wap, bap):
    raise NotImplementedError("write your pallas kernel here")


# speedup vs baseline: 1674.8500x; 1674.8500x over previous
import numpy as np
import jax
import jax.numpy as jnp
from jax.experimental import pallas as pl

NUM_AGENTS = 16
ACTION_DIM = 8
NUM_BELIEF = 120
HIDDEN = 32
HEADS = 4
D = HIDDEN * HEADS
BB = 256


def _np_consts():
    pas = np.zeros((64, 1024), np.float32)
    for s in range(16):
        for h in range(HEADS):
            for d in range(16):
                pas[s * 4 + h, s * 64 + d * 4 + h] = 1.0
    rsum = np.zeros((1024, 64), np.float32)
    for s in range(16):
        for d in range(16):
            for h in range(HEADS):
                rsum[s * 64 + d * 4 + h, d * 4 + h] = 1.0
    q = np.zeros((64, 2048), np.float32)
    for d in range(16):
        for h in range(HEADS):
            for c in range(HIDDEN):
                q[d * 4 + h, d * 128 + h * 32 + c] = 1.0
    r4 = np.zeros((64, 4), np.float32)
    for s in range(16):
        for h in range(HEADS):
            r4[s * 4 + h, h] = 1.0
    q4 = np.zeros((4, 128), np.float32)
    for h in range(HEADS):
        for c in range(HIDDEN):
            q4[h, h * 32 + c] = 1.0
    return pas, rsum, q, r4, q4


_PAS, _RSUM, _Q, _R4, _Q4 = _np_consts()


def _mm(a, b):
    return jnp.dot(a, b, preferred_element_type=jnp.float32)


def _tile16(x):
    return jnp.concatenate([x] * 16, axis=1)


def _attn_full(xs, asd, pas, rsum, q):
    rs = [_mm(x, asd) for x in xs]
    AS = jnp.concatenate([r[:, 0:4] for r in rs], axis=1)
    AD = jnp.concatenate([r[:, 4:8] for r in rs], axis=1)
    E = _mm(AS, pas) + _tile16(AD)
    E = jnp.where(E >= 0, E, 0.2 * E)
    rm = jnp.max(E, axis=1, keepdims=True)
    EX = jnp.exp(E - rm)
    DEN = _mm(EX, rsum)
    ALPHA = EX / (_tile16(DEN) + 1e-16)
    out = None
    for s in range(16):
        bc = _mm(ALPHA[:, 64 * s:64 * s + 64], q)
        term = bc * _tile16(xs[s])
        out = term if out is None else out + term
    return out


def _attn_ego(xs, asd, r4, q4):
    rs = [_mm(x, asd) for x in xs]
    AS = jnp.concatenate([r[:, 0:4] for r in rs], axis=1)
    E = AS + _tile16(rs[0][:, 4:8])
    E = jnp.where(E >= 0, E, 0.2 * E)
    rm = jnp.max(E, axis=1, keepdims=True)
    EX = jnp.exp(E - rm)
    DEN = _mm(EX, r4)
    ALPHA = EX / (_tile16(DEN) + 1e-16)
    out = None
    for s in range(16):
        bc = _mm(ALPHA[:, 4 * s:4 * s + 4], q4)
        term = bc * xs[s]
        out = term if out is None else out + term
    return out


def _body(sig_ref, act_ref, w1_ref, asd1_ref, b1_ref, w2_ref, asd2_ref,
          b2_ref, wv_ref, bv_ref, wo_ref, bo_ref, wlm_ref, blm_ref,
          wap_ref, bap_ref, pas_ref, rsum_ref, q_ref, r4_ref, q4_ref,
          out_ref):
    sig = sig_ref[...]
    act = act_ref[...]
    W1 = w1_ref[...]
    pas = pas_ref[...]
    rsum = rsum_ref[...]
    q = q_ref[...]

    w1a = W1[NUM_BELIEF:, :]
    xs = [_mm(act[:, ACTION_DIM * j:ACTION_DIM * (j + 1)], w1a)
          for j in range(NUM_AGENTS)]
    xs[0] = xs[0] + _mm(sig, W1[:NUM_BELIEF, :])

    o1 = _attn_full(xs, asd1_ref[...], pas, rsum, q)
    h = jnp.maximum(o1 + _tile16(b1_ref[...]), 0.0)

    W2 = w2_ref[...]
    xs2 = [_mm(h[:, 128 * j:128 * (j + 1)], W2) for j in range(NUM_AGENTS)]

    ego = _attn_ego(xs2, asd2_ref[...], r4_ref[...], q4_ref[...])
    ego = jnp.maximum(ego + b2_ref[...], 0.0)

    v = _mm(ego, wv_ref[...]) + bv_ref[...]
    tf = _mm(v, wo_ref[...]) + bo_ref[...]
    mean = _mm(tf, wlm_ref[...]) + blm_ref[...]
    out_ref[...] = _mm(mean, wap_ref[...]) + bap_ref[...]


def _asd(att_src, att_dst):
    src_flat = att_src.reshape(-1)
    dst_flat = att_dst.reshape(-1)
    mask = jnp.asarray(_Q4.T)
    return jnp.concatenate([mask * src_flat[:, None],
                            mask * dst_flat[:, None]], axis=1)


def kernel(signals, neighbor_actions, W1, att_src1, att_dst1, b1, W2,
           att_src2, att_dst2, b2, Wq, bq, Wk, bk, Wv, bv, Wo, bo,
           Wlm, blm, Wap, bap):
    B = signals.shape[0]
    grid = (B // BB,)
    asd1 = _asd(att_src1, att_dst1)
    asd2 = _asd(att_src2, att_dst2)
    consts = [jnp.asarray(c) for c in (_PAS, _RSUM, _Q, _R4, _Q4)]

    def full(a):
        return pl.BlockSpec(a.shape, lambda i: (0,) * a.ndim)

    weights = [W1, asd1, b1.reshape(1, D), W2, asd2, b2.reshape(1, D),
               Wv, bv.reshape(1, D), Wo, bo.reshape(1, D),
               Wlm, blm.reshape(1, -1), Wap, bap.reshape(1, -1)] + consts

    return pl.pallas_call(
        _body,
        grid=grid,
        in_specs=[pl.BlockSpec((BB, NUM_BELIEF), lambda i: (i, 0)),
                  pl.BlockSpec((BB, NUM_AGENTS * ACTION_DIM), lambda i: (i, 0))]
                 + [full(w) for w in weights],
        out_specs=pl.BlockSpec((BB, ACTION_DIM * NUM_AGENTS), lambda i: (i, 0)),
        out_shape=jax.ShapeDtypeStruct((B, ACTION_DIM * NUM_AGENTS),
                                       jnp.float32),
    )(signals, neighbor_actions, *weights)


# speedup vs baseline: 2105.9696x; 1.2574x over previous
import numpy as np
import jax
import jax.numpy as jnp
from jax.experimental import pallas as pl

NUM_AGENTS = 16
ACTION_DIM = 8
NUM_BELIEF = 120
HIDDEN = 32
HEADS = 4
D = HIDDEN * HEADS
BB = 1024


def _np_consts():
    p2 = np.zeros((128, 1024), np.float32)
    for s in range(16):
        for d in range(16):
            for h in range(HEADS):
                p2[s * 8 + h, s * 64 + d * 4 + h] = 1.0
                p2[d * 8 + 4 + h, s * 64 + d * 4 + h] = 1.0
    p20 = np.zeros((128, 64), np.float32)
    for s in range(16):
        for h in range(HEADS):
            p20[s * 8 + h, s * 4 + h] = 1.0
            p20[4 + h, s * 4 + h] += 1.0
    rsum = np.zeros((1024, 64), np.float32)
    for s in range(16):
        for d in range(16):
            for h in range(HEADS):
                rsum[s * 64 + d * 4 + h, d * 4 + h] = 1.0
    q = np.zeros((64, 2048), np.float32)
    for d in range(16):
        for h in range(HEADS):
            for c in range(HIDDEN):
                q[d * 4 + h, d * 128 + h * 32 + c] = 1.0
    q64 = np.zeros((64, 2048), np.float32)
    for s in range(16):
        for h in range(HEADS):
            for c in range(HIDDEN):
                q64[s * 4 + h, s * 128 + h * 32 + c] = 1.0
    r4 = np.zeros((64, 4), np.float32)
    for s in range(16):
        for h in range(HEADS):
            r4[s * 4 + h, h] = 1.0
    q4 = np.zeros((4, 128), np.float32)
    for h in range(HEADS):
        for c in range(HIDDEN):
            q4[h, h * 32 + c] = 1.0
    return p2, p20, rsum, q, q64, r4, q4


_P2, _P20, _RSUM, _Q, _Q64, _R4, _Q4 = _np_consts()


def _mm(a, b):
    return jnp.dot(a, b, preferred_element_type=jnp.float32)


def _body(sig_ref, act_ref, w1blk_ref, w1s_ref, asd1blk_ref, b1_ref,
          w2_ref, asd2_ref, b2_ref, wtail_ref, btail_ref, p2_ref, qb_ref,
          rsumb_ref, p20_ref, q64b_ref, r4b_ref, q4_ref, out_ref):
    sig = sig_ref[...]
    act = act_ref[...]

    xs_all = _mm(act, w1blk_ref[...])
    sigp = _mm(sig, w1s_ref[...])
    xs_all = jnp.concatenate([xs_all[:, :D] + sigp, xs_all[:, D:]], axis=1)

    asad = _mm(xs_all, asd1blk_ref[...])
    E = _mm(asad, p2_ref[...])
    E = jnp.where(E >= 0, E, 0.2 * E)
    rm = jnp.max(E, axis=1, keepdims=True)
    EX = jnp.exp(E - rm)
    exb = EX.astype(jnp.bfloat16)
    qb = qb_ref[...]
    DEN = _mm(exb, rsumb_ref[...])
    DENbc = _mm(DEN.astype(jnp.bfloat16), qb)
    acc = [None] * NUM_AGENTS
    for s in range(NUM_AGENTS):
        bc = _mm(exb[:, 64 * s:64 * s + 64], qb)
        xsrc = xs_all[:, D * s:D * s + D]
        for d in range(NUM_AGENTS):
            term = bc[:, D * d:D * d + D] * xsrc
            acc[d] = term if acc[d] is None else acc[d] + term
    ACC = jnp.concatenate(acc, axis=0)
    DENr = jnp.concatenate([DENbc[:, D * d:D * d + D]
                            for d in range(NUM_AGENTS)], axis=0)
    HS = jnp.maximum(ACC / (DENr + 1e-16) + b1_ref[...], 0.0)

    XS2 = _mm(HS, w2_ref[...])
    ASAD2 = _mm(XS2, asd2_ref[...])
    sc2 = jnp.concatenate([ASAD2[s * BB:(s + 1) * BB]
                           for s in range(NUM_AGENTS)], axis=1)
    E0 = _mm(sc2, p20_ref[...])
    E0 = jnp.where(E0 >= 0, E0, 0.2 * E0)
    rm0 = jnp.max(E0, axis=1, keepdims=True)
    EX0 = jnp.exp(E0 - rm0)
    exb0 = EX0.astype(jnp.bfloat16)
    DEN0 = _mm(exb0, r4b_ref[...])
    den0bc = _mm(DEN0, q4_ref[...])
    ego_bc = _mm(exb0, q64b_ref[...])
    ego = None
    for s in range(NUM_AGENTS):
        term = ego_bc[:, D * s:D * s + D] * XS2[s * BB:(s + 1) * BB]
        ego = term if ego is None else ego + term
    ego = jnp.maximum(ego / (den0bc + 1e-16) + b2_ref[...], 0.0)

    out_ref[...] = _mm(ego, wtail_ref[...]) + btail_ref[...]


def _asd(att_src, att_dst):
    src_flat = att_src.reshape(-1)
    dst_flat = att_dst.reshape(-1)
    mask = jnp.asarray(_Q4.T)
    return jnp.concatenate([mask * src_flat[:, None],
                            mask * dst_flat[:, None]], axis=1)


def kernel(signals, neighbor_actions, W1, att_src1, att_dst1, b1, W2,
           att_src2, att_dst2, b2, Wq, bq, Wk, bk, Wv, bv, Wo, bo,
           Wlm, blm, Wap, bap):
    B = signals.shape[0]
    grid = (B // BB,)
    eye16 = jnp.eye(NUM_AGENTS, dtype=jnp.float32)
    w1blk = jnp.kron(eye16, W1[NUM_BELIEF:, :])
    asd1blk = jnp.kron(eye16, _asd(att_src1, att_dst1))
    asd2 = _asd(att_src2, att_dst2)
    m1 = Wlm @ Wap
    m2 = Wo @ m1
    wtail = Wv @ m2
    btail = bv @ m2 + bo @ m1 + blm @ Wap + bap
    consts = [jnp.asarray(_P2), jnp.asarray(_Q, jnp.bfloat16),
              jnp.asarray(_RSUM, jnp.bfloat16), jnp.asarray(_P20),
              jnp.asarray(_Q64, jnp.bfloat16), jnp.asarray(_R4, jnp.bfloat16),
              jnp.asarray(_Q4)]

    def full(a):
        return pl.BlockSpec(a.shape, lambda i: (0,) * a.ndim)

    weights = [w1blk, W1[:NUM_BELIEF, :], asd1blk, b1.reshape(1, D),
               W2, asd2, b2.reshape(1, D),
               wtail, btail.reshape(1, -1)] + consts

    return pl.pallas_call(
        _body,
        grid=grid,
        in_specs=[pl.BlockSpec((BB, NUM_BELIEF), lambda i: (i, 0)),
                  pl.BlockSpec((BB, NUM_AGENTS * ACTION_DIM), lambda i: (i, 0))]
                 + [full(w) for w in weights],
        out_specs=pl.BlockSpec((BB, ACTION_DIM * NUM_AGENTS), lambda i: (i, 0)),
        out_shape=jax.ShapeDtypeStruct((B, ACTION_DIM * NUM_AGENTS),
                                       jnp.float32),
    )(signals, neighbor_actions, *weights)
